# Initial kernel scaffold; baseline (speedup 1.0000x reference)
import jax
import jax.numpy as jnp
from jax.experimental import pallas as pl


def kernel(x, W_r, b_r, W_ap, b_ap, W_pa, b_pa):
    raise NotImplementedError("write your pallas kernel here")


# speedup vs baseline: 1.5573x; 1.5573x over previous
import functools

import jax
import jax.numpy as jnp
from jax.experimental import pallas as pl
from jax.experimental.pallas import tpu as pltpu

_TOKENS = 8192
_D = 1024
_BM = 512


def _fused_body(x_ref, wr_ref, br_ref, wap_ref, bap_ref, wpa_ref, bpa_ref,
                out_ref):
    x32 = x_ref[...]
    logits = jax.lax.dot_general(
        x32, wr_ref[...], (((1,), (0,)), ((), ())),
        preferred_element_type=jnp.float32) + br_ref[...]
    pred = jax.nn.sigmoid(logits) > 0.5
    xb = x32.astype(jnp.bfloat16)
    oap = jnp.dot(xb, wap_ref[...], preferred_element_type=jnp.float32)
    opa = jnp.dot(xb, wpa_ref[...], preferred_element_type=jnp.float32)
    oap = oap + bap_ref[...]
    opa = opa + bpa_ref[...]
    out_ref[...] = jnp.where(pred, oap, opa)


@functools.partial(jax.jit, static_argnames=("interpret",))
def _run(x, W_r, b_r, W_ap, b_ap, W_pa, b_pa, interpret=False):
    grid = (_TOKENS // _BM,)
    full = lambda shape: pl.BlockSpec(shape, lambda i: (0, 0))
    return pl.pallas_call(
        _fused_body,
        grid=grid,
        in_specs=[
            pl.BlockSpec((_BM, _D), lambda i: (i, 0)),
            full((_D, 1)),
            full((1, 1)),
            full((_D, _D)),
            full((1, _D)),
            full((_D, _D)),
            full((1, _D)),
        ],
        out_specs=pl.BlockSpec((_BM, _D), lambda i: (i, 0)),
        out_shape=jax.ShapeDtypeStruct((_TOKENS, _D), jnp.float32),
        compiler_params=pltpu.CompilerParams(
            dimension_semantics=("arbitrary",)),
        interpret=interpret,
    )(x, W_r, b_r.reshape(1, 1), W_ap.astype(jnp.bfloat16),
      b_ap.reshape(1, _D), W_pa.astype(jnp.bfloat16), b_pa.reshape(1, _D))


def kernel(x, W_r, b_r, W_ap, b_ap, W_pa, b_pa):
    return _run(x, W_r, b_r, W_ap, b_ap, W_pa, b_pa)
